# Initial kernel scaffold; baseline (speedup 1.0000x reference)
import jax
import jax.numpy as jnp
from jax.experimental import pallas as pl


def kernel(node_context, original_data, cell_context, high_mask, low_mask, init_w, Wh, bh, Wv, bv, Wq, Wref, v, w_low):
    raise NotImplementedError("write your pallas kernel here")


# speedup vs baseline: 3.6294x; 3.6294x over previous
import jax
import jax.numpy as jnp
from jax.experimental import pallas as pl
from jax.experimental.pallas import tpu as pltpu

_C = 10.0
_BIG = 1e9


def _low_logits_body(nc_ref, w_ref, out_ref):
    nc = nc_ref[...]
    nb, ns, np_, ne = nc.shape
    vrep = jnp.broadcast_to(w_ref[...][0][:, None], (ne, 128))
    lane_p = jax.lax.broadcasted_iota(jnp.int32, (nb, ns, np_), 2)
    acc = None
    for pp in range(np_):
        rp = jnp.dot(nc[:, :, pp, :].reshape(nb * ns, ne), vrep,
                     preferred_element_type=jnp.float32)
        ex = jnp.max(rp.reshape(nb, ns, 128), axis=-1)
        contrib = jnp.where(lane_p == pp, ex[:, :, None], 0.0)
        acc = contrib if acc is None else acc + contrib
    out_ref[...] = _C * jnp.tanh(acc)


def _decoder_body(cc_ref, lt_ref, ox_ref, oy_ref, lm_ref, hm_ref,
                  gh_ref, gl_ref, hbar_ref, q0_ref, wv_ref, bv_ref,
                  wq_ref, wref_ref, v_ref,
                  hlp_ref, llp_ref, hr_ref, lr_ref, ha_ref, la_ref,
                  reft_ref):
    b, s, e = cc_ref.shape
    p = lt_ref.shape[-1]
    h = wq_ref.shape[-1]
    f32 = jnp.float32

    def rep(x):
        return jnp.broadcast_to(x, (b, s))

    def rep_p(x):
        return jnp.broadcast_to(x, (b, p))

    cc = cc_ref[...]
    wv = wv_ref[...]
    bv = bv_ref[...]
    h_bar = hbar_ref[...]
    query0 = q0_ref[...]

    reft_ref[...] = jnp.dot(cc.reshape(b * s, e), wref_ref[...],
                            preferred_element_type=f32).reshape(b, s, h)

    wq = wq_ref[...]
    vrep = jnp.broadcast_to(v_ref[...][0][:, None], (h, 128))
    wv1 = wv[:e]
    wv2 = wv[e:]
    lt = lt_ref[...]
    lm = lm_ref[...]
    ox = ox_ref[...]
    oy = oy_ref[...]
    lane_s = jax.lax.broadcasted_iota(jnp.int32, (b, s), 1)
    lane_p = jax.lax.broadcasted_iota(jnp.int32, (b, p), 1)
    hm = hm_ref[...]

    def step(i, carry):
        (query, mask, init_h,
         lp_l, hr_l, llp_l, lr_l, lnx_l, lny_l, ha, la) = carry
        q = jnp.dot(query, wq, preferred_element_type=f32)
        t = jnp.tanh(reft_ref[...] + q[:, None, :])
        tv = jnp.max(jnp.dot(t.reshape(b * s, h), vrep,
                             preferred_element_type=f32).reshape(b, s, 128),
                     axis=-1)
        u = _C * jnp.tanh(tv)
        um = u - _BIG * mask
        z = um + gh_ref[i]
        mz = rep(jnp.max(z, axis=-1, keepdims=True))
        idx = rep(jnp.min(jnp.where(z >= mz, lane_s, s),
                          axis=-1, keepdims=True))
        idx = jnp.where(i == 0, 0, idx)
        oh = (lane_s == idx).astype(f32)
        mxu = rep(jnp.max(um, axis=-1, keepdims=True))
        lse = rep(jnp.log(jnp.sum(jnp.exp(um - mxu),
                                  axis=-1, keepdims=True)))
        lp = (rep(jnp.sum(um * oh, axis=-1, keepdims=True)) - mxu) - lse

        ohc = oh[:, :, None]
        hsel = jnp.sum(cc * ohc, axis=1)
        lt_s = jnp.sum(lt * ohc, axis=1)
        lm_s = jnp.sum(lm * ohc, axis=1)
        ox_s = jnp.sum(ox * ohc, axis=1)
        oy_s = jnp.sum(oy * ohc, axis=1)

        low_u = lt_s - _BIG * lm_s
        zl = low_u + gl_ref[i]
        mzl = rep_p(jnp.max(zl, axis=-1, keepdims=True))
        lidx = jnp.min(jnp.where(zl >= mzl, lane_p, p),
                       axis=-1, keepdims=True)
        lidx_p = jnp.broadcast_to(lidx, (b, p))
        lidx_s = jnp.broadcast_to(lidx, (b, s))
        ohl = (lane_p == lidx_p).astype(f32)
        mxl = rep_p(jnp.max(low_u, axis=-1, keepdims=True))
        lsel = jnp.log(jnp.sum(jnp.exp(low_u - mxl), axis=-1, keepdims=True))
        llp_i = (rep(jnp.sum(low_u * ohl, axis=-1, keepdims=True))
                 - rep(mxl[:, 0:1])) - rep(lsel)

        inx = rep(ox_s[:, 0:1])
        iny = rep(oy_s[:, 0:1])
        lnx = rep(jnp.sum(ox_s * ohl, axis=-1, keepdims=True))
        lny = rep(jnp.sum(oy_s * ohl, axis=-1, keepdims=True))
        ohm1 = (lane_s == i - 1).astype(f32)
        last_x = rep(jnp.sum(lnx_l * ohm1, axis=-1, keepdims=True))
        last_y = rep(jnp.sum(lny_l * ohm1, axis=-1, keepdims=True))
        low_r = jnp.sqrt((lnx - inx) ** 2 + (lny - iny) ** 2 + 1e-12)
        cell_r = jnp.sqrt((last_x - inx) ** 2 + (last_y - iny) ** 2 + 1e-12)

        mask = jnp.minimum(mask + oh, 1.0)
        init_h = jnp.where(i == 0, hsel, init_h)
        h_rest = (jnp.dot(init_h, wv1, preferred_element_type=f32)
                  + jnp.dot(hsel, wv2, preferred_element_type=f32) + bv)
        query = h_bar + h_rest

        here = lane_s == i
        lp_l = jnp.where(here, lp, lp_l)
        hr_l = jnp.where(here, cell_r, hr_l)
        llp_l = jnp.where(here, llp_i, llp_l)
        lr_l = jnp.where(here, low_r, lr_l)
        lnx_l = jnp.where(here, lnx, lnx_l)
        lny_l = jnp.where(here, lny, lny_l)
        ha = jnp.where(here, idx, ha)
        la = jnp.where(here, lidx_s, la)
        return (query, mask, init_h,
                lp_l, hr_l, llp_l, lr_l, lnx_l, lny_l, ha, la)

    zf = hm * 0.0
    zi = lane_s * 0
    init_h0 = cc[:, 0, :] * 0.0
    carry0 = (query0, hm, init_h0,
              zf, zf, zf, zf, zf, zf, zi, zi)
    (_, _, _, lp_l, hr_l, llp_l, lr_l, _, _, ha, la) = jax.lax.fori_loop(
        0, s, step, carry0)
    hlp_ref[...] = jnp.sum(lp_l, axis=-1, keepdims=True)
    llp_ref[...] = jnp.sum(llp_l, axis=-1, keepdims=True)
    hr_ref[...] = jnp.sum(hr_l, axis=-1, keepdims=True)
    lr_ref[...] = jnp.sum(lr_l, axis=-1, keepdims=True)
    ha_ref[...] = ha
    la_ref[...] = la


def kernel(node_context, original_data, cell_context, high_mask, low_mask,
           init_w, Wh, bh, Wv, bv, Wq, Wref, v, w_low):
    b, s, e = cell_context.shape
    p = node_context.shape[2]
    f32 = jnp.float32

    skey = jax.random.key(42)
    ii = jnp.arange(s, dtype=jnp.int32)
    kh = jax.vmap(lambda i: jax.random.fold_in(skey, i))(ii)
    kl = jax.vmap(lambda i: jax.random.fold_in(skey, 10000 + i))(ii)
    g_high = jax.vmap(lambda k: jax.random.gumbel(k, (b, s), f32))(kh)
    g_low = jax.vmap(lambda k: jax.random.gumbel(k, (b, p), f32))(kl)

    h_mean = jnp.mean(cell_context, axis=1)
    h_bar = h_mean @ Wh + bh
    h_rest0 = init_w @ Wv + bv
    query0 = h_bar + h_rest0[None, :]

    bblk = 8
    low_t = pl.pallas_call(
        _low_logits_body,
        grid=(b // bblk,),
        in_specs=[pl.BlockSpec((bblk, s, p, e), lambda i: (i, 0, 0, 0)),
                  pl.BlockSpec((1, e), lambda i: (0, 0))],
        out_specs=pl.BlockSpec((bblk, s, p), lambda i: (i, 0, 0)),
        out_shape=jax.ShapeDtypeStruct((b, s, p), f32),
    )(node_context, w_low.reshape(1, e))

    ox = original_data[..., 0]
    oy = original_data[..., 1]
    out_shape = [
        jax.ShapeDtypeStruct((b, 1), f32),
        jax.ShapeDtypeStruct((b, 1), f32),
        jax.ShapeDtypeStruct((b, 1), f32),
        jax.ShapeDtypeStruct((b, 1), f32),
        jax.ShapeDtypeStruct((b, s), jnp.int32),
        jax.ShapeDtypeStruct((b, s), jnp.int32),
    ]
    hlp, llp, hr, lr, ha, la = pl.pallas_call(
        _decoder_body,
        out_shape=out_shape,
        scratch_shapes=[pltpu.VMEM((b, s, e), f32)],
    )(cell_context, low_t, ox, oy, low_mask, high_mask,
      g_high, g_low, h_bar, query0, Wv, bv.reshape(1, e),
      Wq, Wref, v.reshape(1, e))

    return (hlp[:, 0], llp[:, 0], hr[:, 0], lr[:, 0], ha, la)


# speedup vs baseline: 5.8704x; 1.6174x over previous
import jax
import jax.numpy as jnp
from jax.experimental import pallas as pl
from jax.experimental.pallas import tpu as pltpu

_C = 10.0
_BIG = 1e9


def _low_logits_body(nc_ref, w_ref, out_ref):
    nc = nc_ref[...]
    nb, ns, np_, ne = nc.shape
    vrep = jnp.broadcast_to(w_ref[...][0][:, None], (ne, 128))
    lane_p = jax.lax.broadcasted_iota(jnp.int32, (nb, ns, np_), 2)
    acc = None
    for pp in range(np_):
        rp = jnp.dot(nc[:, :, pp, :].reshape(nb * ns, ne), vrep,
                     preferred_element_type=jnp.float32)
        ex = jnp.max(rp.reshape(nb, ns, 128), axis=-1)
        contrib = jnp.where(lane_p == pp, ex[:, :, None], 0.0)
        acc = contrib if acc is None else acc + contrib
    out_ref[...] = _C * jnp.tanh(acc)


def _attention_body(cc_ref, hbar_ref, q0_ref, wv_ref, bv_ref, wq_ref,
                    wref_ref, v_ref, u0_ref, u_all_ref, reft_ref, qall_ref):
    b, s, e = cc_ref.shape
    h = wq_ref.shape[-1]
    f32 = jnp.float32
    cc = cc_ref[...]
    cc2d = cc.reshape(b * s, e)
    wv = wv_ref[...]
    bv = bv_ref[...]
    wq = wq_ref[...]

    reft_ref[...] = jnp.dot(cc2d, wref_ref[...],
                            preferred_element_type=f32).reshape(b, s, h)
    hv1 = jnp.dot(cc[:, 0, :], wv[:e], preferred_element_type=f32)
    hv2 = jnp.dot(cc2d, wv[e:], preferred_element_type=f32).reshape(b, s, e)
    qall = hbar_ref[...][:, None, :] + ((hv1[:, None, :] + hv2) + bv[0][None, None, :])
    qall_ref[...] = jnp.dot(qall.reshape(b * s, e), wq,
                            preferred_element_type=f32).reshape(b, s, h)

    vrep = jnp.broadcast_to(v_ref[...][0][:, None], (h, 128))
    reft = reft_ref[...]

    def att(qrow):
        t = jnp.tanh(reft + qrow[:, None, :])
        tv = jnp.max(jnp.dot(t.reshape(b * s, h), vrep,
                             preferred_element_type=f32).reshape(b, s, 128),
                     axis=-1)
        return _C * jnp.tanh(tv)

    q0 = jnp.dot(q0_ref[...], wq, preferred_element_type=f32)
    u0_ref[...] = att(q0)
    qall3 = qall_ref[...]
    for a in range(s):
        u_all_ref[:, a, :] = att(qall3[:, a, :])


def _decoder_body(u0_ref, u_all_ref, lt_ref, ox_ref, oy_ref, lm_ref, hm_ref,
                  gh_ref, gl_ref,
                  hlp_ref, llp_ref, hr_ref, lr_ref, ha_ref, la_ref):
    b, s = u0_ref.shape
    p = lt_ref.shape[-1]
    f32 = jnp.float32

    def rep(x):
        return jnp.broadcast_to(x, (b, s))

    def rep_p(x):
        return jnp.broadcast_to(x, (b, p))

    u0 = u0_ref[...]
    u_all = u_all_ref[...]
    lt = lt_ref[...]
    lm = lm_ref[...]
    ox = ox_ref[...]
    oy = oy_ref[...]
    lane_s = jax.lax.broadcasted_iota(jnp.int32, (b, s), 1)
    lane_p = jax.lax.broadcasted_iota(jnp.int32, (b, p), 1)
    hm = hm_ref[...]

    def step(i, carry):
        (oh_prev, mask,
         lp_l, hr_l, llp_l, lr_l, lnx_l, lny_l, ha, la) = carry
        uu = jnp.sum(u_all * oh_prev[:, :, None], axis=1)
        u = jnp.where(i == 0, u0, uu)
        um = u - _BIG * mask
        z = um + gh_ref[i]
        mz = rep(jnp.max(z, axis=-1, keepdims=True))
        idx = rep(jnp.min(jnp.where(z >= mz, lane_s, s),
                          axis=-1, keepdims=True))
        idx = jnp.where(i == 0, 0, idx)
        oh = (lane_s == idx).astype(f32)
        mxu = rep(jnp.max(um, axis=-1, keepdims=True))
        lse = rep(jnp.log(jnp.sum(jnp.exp(um - mxu),
                                  axis=-1, keepdims=True)))
        lp = (rep(jnp.sum(um * oh, axis=-1, keepdims=True)) - mxu) - lse

        ohc = oh[:, :, None]
        lt_s = jnp.sum(lt * ohc, axis=1)
        lm_s = jnp.sum(lm * ohc, axis=1)
        ox_s = jnp.sum(ox * ohc, axis=1)
        oy_s = jnp.sum(oy * ohc, axis=1)

        low_u = lt_s - _BIG * lm_s
        zl = low_u + gl_ref[i]
        mzl = rep_p(jnp.max(zl, axis=-1, keepdims=True))
        lidx = jnp.min(jnp.where(zl >= mzl, lane_p, p),
                       axis=-1, keepdims=True)
        lidx_p = jnp.broadcast_to(lidx, (b, p))
        lidx_s = jnp.broadcast_to(lidx, (b, s))
        ohl = (lane_p == lidx_p).astype(f32)
        mxl = rep_p(jnp.max(low_u, axis=-1, keepdims=True))
        lsel = jnp.log(jnp.sum(jnp.exp(low_u - mxl), axis=-1, keepdims=True))
        llp_i = (rep(jnp.sum(low_u * ohl, axis=-1, keepdims=True))
                 - rep(mxl[:, 0:1])) - rep(lsel)

        inx = rep(ox_s[:, 0:1])
        iny = rep(oy_s[:, 0:1])
        lnx = rep(jnp.sum(ox_s * ohl, axis=-1, keepdims=True))
        lny = rep(jnp.sum(oy_s * ohl, axis=-1, keepdims=True))
        ohm1 = (lane_s == i - 1).astype(f32)
        last_x = rep(jnp.sum(lnx_l * ohm1, axis=-1, keepdims=True))
        last_y = rep(jnp.sum(lny_l * ohm1, axis=-1, keepdims=True))
        low_r = jnp.sqrt((lnx - inx) ** 2 + (lny - iny) ** 2 + 1e-12)
        cell_r = jnp.sqrt((last_x - inx) ** 2 + (last_y - iny) ** 2 + 1e-12)

        mask = jnp.minimum(mask + oh, 1.0)

        here = lane_s == i
        lp_l = jnp.where(here, lp, lp_l)
        hr_l = jnp.where(here, cell_r, hr_l)
        llp_l = jnp.where(here, llp_i, llp_l)
        lr_l = jnp.where(here, low_r, lr_l)
        lnx_l = jnp.where(here, lnx, lnx_l)
        lny_l = jnp.where(here, lny, lny_l)
        ha = jnp.where(here, idx, ha)
        la = jnp.where(here, lidx_s, la)
        return (oh, mask,
                lp_l, hr_l, llp_l, lr_l, lnx_l, lny_l, ha, la)

    zf = hm * 0.0
    zi = lane_s * 0
    carry0 = (zf, hm,
              zf, zf, zf, zf, zf, zf, zi, zi)
    (_, _, lp_l, hr_l, llp_l, lr_l, _, _, ha, la) = jax.lax.fori_loop(
        0, s, step, carry0)
    hlp_ref[...] = jnp.sum(lp_l, axis=-1, keepdims=True)
    llp_ref[...] = jnp.sum(llp_l, axis=-1, keepdims=True)
    hr_ref[...] = jnp.sum(hr_l, axis=-1, keepdims=True)
    lr_ref[...] = jnp.sum(lr_l, axis=-1, keepdims=True)
    ha_ref[...] = ha
    la_ref[...] = la


def kernel(node_context, original_data, cell_context, high_mask, low_mask,
           init_w, Wh, bh, Wv, bv, Wq, Wref, v, w_low):
    b, s, e = cell_context.shape
    p = node_context.shape[2]
    f32 = jnp.float32

    skey = jax.random.key(42)
    ii = jnp.arange(s, dtype=jnp.int32)
    kh = jax.vmap(lambda i: jax.random.fold_in(skey, i))(ii)
    kl = jax.vmap(lambda i: jax.random.fold_in(skey, 10000 + i))(ii)
    g_high = jax.vmap(lambda k: jax.random.gumbel(k, (b, s), f32))(kh)
    g_low = jax.vmap(lambda k: jax.random.gumbel(k, (b, p), f32))(kl)

    h_mean = jnp.mean(cell_context, axis=1)
    h_bar = h_mean @ Wh + bh
    h_rest0 = init_w @ Wv + bv
    query0 = h_bar + h_rest0[None, :]

    bblk = 8
    low_t = pl.pallas_call(
        _low_logits_body,
        grid=(b // bblk,),
        in_specs=[pl.BlockSpec((bblk, s, p, e), lambda i: (i, 0, 0, 0)),
                  pl.BlockSpec((1, e), lambda i: (0, 0))],
        out_specs=pl.BlockSpec((bblk, s, p), lambda i: (i, 0, 0)),
        out_shape=jax.ShapeDtypeStruct((b, s, p), f32),
    )(node_context, w_low.reshape(1, e))

    u0, u_all = pl.pallas_call(
        _attention_body,
        out_shape=[jax.ShapeDtypeStruct((b, s), f32),
                   jax.ShapeDtypeStruct((b, s, s), f32)],
        scratch_shapes=[pltpu.VMEM((b, s, e), f32),
                        pltpu.VMEM((b, s, e), f32)],
    )(cell_context, h_bar, query0, Wv, bv.reshape(1, e), Wq, Wref,
      v.reshape(1, e))

    ox = original_data[..., 0]
    oy = original_data[..., 1]
    out_shape = [
        jax.ShapeDtypeStruct((b, 1), f32),
        jax.ShapeDtypeStruct((b, 1), f32),
        jax.ShapeDtypeStruct((b, 1), f32),
        jax.ShapeDtypeStruct((b, 1), f32),
        jax.ShapeDtypeStruct((b, s), jnp.int32),
        jax.ShapeDtypeStruct((b, s), jnp.int32),
    ]
    hlp, llp, hr, lr, ha, la = pl.pallas_call(
        _decoder_body,
        out_shape=out_shape,
    )(u0, u_all, low_t, ox, oy, low_mask, high_mask, g_high, g_low)

    return (hlp[:, 0], llp[:, 0], hr[:, 0], lr[:, 0], ha, la)
